# baseline (speedup 1.0000x reference)
import functools

import jax
import jax.numpy as jnp
from jax import lax
from jax.experimental import pallas as pl
from jax.experimental.pallas import tpu as pltpu
from jax.experimental.pallas import tpu_sc as plsc

B = 16384
D = 32
L = 16
NC, NS = 2, 16
NW = NC * NS
BPW = B // NW
GB = 16
NG = BPW // GB
W = 128
HS = 16
TAIL0 = 999872
RTAIL = TAIL0 + 64

_mesh = plsc.VectorSubcoreMesh(core_axis_name="c", subcore_axis_name="s")


@functools.partial(
    pl.kernel,
    mesh=_mesh,
    compiler_params=pltpu.CompilerParams(needs_layout_passes=False),
    out_type=jax.ShapeDtypeStruct((B,), jnp.float32),
    scratch_types=[
        pltpu.VMEM((BPW,), jnp.int32),
        pltpu.VMEM((BPW,), jnp.int32),
        pltpu.VMEM((GB + 8, D, W), jnp.float32),
        pltpu.VMEM((D, W), jnp.float32),
        pltpu.VMEM((D, W), jnp.float32),
        pltpu.VMEM((D, L), jnp.float32),
        pltpu.VMEM((D, L), jnp.float32),
        pltpu.VMEM((BPW,), jnp.float32),
        pltpu.SemaphoreType.DMA,
        pltpu.SemaphoreType.DMA,
        pltpu.SemaphoreType.DMA,
    ],
)
def _mf_score(uidx_hbm, iidx_hbm, ut_hbm, it_hbm, utail_hbm, itail_hbm,
              out_hbm, uidx_v, iidx_v, buf_v, utail_v, itail_v,
              upan_v, ipan_v, out_v, usem, isem, xsem):
    wid = lax.axis_index("s") * NC + lax.axis_index("c")
    base = wid * BPW

    pltpu.async_copy(uidx_hbm.at[pl.ds(base, BPW)], uidx_v, xsem).wait()
    pltpu.async_copy(iidx_hbm.at[pl.ds(base, BPW)], iidx_v, xsem).wait()
    pltpu.async_copy(utail_hbm, utail_v, xsem).wait()
    pltpu.async_copy(itail_hbm, itail_v, xsem).wait()

    lane = lax.broadcasted_iota(jnp.int32, (L,), 0)
    half = jnp.bitwise_and(lane, 7) + HS

    def win_q(r):
        q = lax.shift_right_logical(r, 7)
        return jnp.where(r >= RTAIL, 0, q)

    def fire_u(gidx):
        rvec = uidx_v[pl.ds(gidx * GB, L)]
        for l in range(GB):
            qoff = pl.multiple_of(win_q(rvec[l]) * W, W)
            pltpu.async_copy(ut_hbm.at[:, pl.ds(qoff, W)],
                             buf_v.at[l], usem)

    def fire_i_half(rvec, h):
        for l in range(8):
            qoff = pl.multiple_of(win_q(rvec[h * 8 + l]) * W, W)
            pltpu.async_copy(it_hbm.at[:, pl.ds(qoff, W)],
                             buf_v.at[HS + l], isem)

    def drain(sem, n):
        for _ in range(n):
            pltpu.make_async_copy(ut_hbm.at[:, pl.ds(0, W)],
                                  buf_v.at[0], sem).wait()

    def gather_cols(slot_vec, tail_v, rvec, c):
        cvec = jnp.full((L,), c, jnp.int32)
        wvec = jnp.bitwise_and(rvec, W - 1)
        tcol = jnp.clip(rvec - TAIL0, 0, W - 1)
        win = plsc.load_gather(buf_v, [slot_vec, cvec, wvec])
        tl = plsc.load_gather(tail_v, [cvec, tcol])
        return jnp.where(rvec >= RTAIL, tl, win)

    fire_u(0)

    def group_body(g, carry):
        urvec = uidx_v[pl.ds(g * GB, L)]
        irvec = iidx_v[pl.ds(g * GB, L)]

        fire_i_half(irvec, 0)
        drain(usem, GB)
        for c in range(D):
            upan_v[c, pl.ds(0, L)] = gather_cols(lane, utail_v, urvec, c)
        fire_u(jnp.minimum(g + 1, NG - 1))

        drain(isem, 8)
        for c in range(D):
            ipan_v[c, pl.ds(0, L)] = gather_cols(half, itail_v, irvec, c)
        fire_i_half(irvec, 1)
        drain(isem, 8)

        hi = lane >= 8
        acc = jnp.zeros((L,), jnp.float32)
        for c in range(D):
            vb = gather_cols(half, itail_v, irvec, c)
            iv = jnp.where(hi, vb, ipan_v[c, pl.ds(0, L)])
            acc = acc + upan_v[c, pl.ds(0, L)] * iv
        out_v[pl.ds(g * GB, L)] = acc
        return carry

    lax.fori_loop(0, NG, group_body, 0)
    drain(usem, GB)
    pltpu.sync_copy(out_v, out_hbm.at[pl.ds(base, BPW)])


def kernel(x, uid_table, iid_table):
    uidx = x[:, 0]
    iidx = x[:, 1]
    utail = lax.slice(uid_table, (TAIL0, 0), (TAIL0 + W, D)).T
    itail = lax.slice(iid_table, (TAIL0, 0), (TAIL0 + W, D)).T
    return _mf_score(uidx, iidx, uid_table.T, iid_table.T, utail, itail)


# speedup vs baseline: 1.0579x; 1.0579x over previous
import functools

import jax
import jax.numpy as jnp
from jax import lax
from jax.experimental import pallas as pl
from jax.experimental.pallas import tpu as pltpu
from jax.experimental.pallas import tpu_sc as plsc

B = 16384
D = 32
L = 16
NC, NS = 2, 16
NW = NC * NS
BPW = B // NW
GB = 16
NG = BPW // GB
W = 128
TAIL0 = 999872
RTAIL = TAIL0 + 64

_mesh = plsc.VectorSubcoreMesh(core_axis_name="c", subcore_axis_name="s")


@functools.partial(
    pl.kernel,
    mesh=_mesh,
    compiler_params=pltpu.CompilerParams(needs_layout_passes=False),
    out_type=jax.ShapeDtypeStruct((B,), jnp.float32),
    scratch_types=[
        pltpu.VMEM((BPW,), jnp.int32),
        pltpu.VMEM((BPW,), jnp.int32),
        pltpu.VMEM((GB, D, W), jnp.float32),
        pltpu.VMEM((D, W), jnp.float32),
        pltpu.VMEM((D, W), jnp.float32),
        pltpu.VMEM((D, L), jnp.float32),
        pltpu.VMEM((D, L), jnp.float32),
        pltpu.VMEM((BPW,), jnp.float32),
        pltpu.SemaphoreType.DMA,
        pltpu.SemaphoreType.DMA,
    ],
)
def _mf_score(uidx_hbm, iidx_hbm, ut_hbm, it_hbm, utail_hbm, itail_hbm,
              out_hbm, uidx_v, iidx_v, buf_v, utail_v,
              itail_v, upan_v, ipan_v, out_v, dsem, xsem):
    wid = lax.axis_index("s") * NC + lax.axis_index("c")
    base = wid * BPW

    pltpu.async_copy(uidx_hbm.at[pl.ds(base, BPW)], uidx_v, xsem).wait()
    pltpu.async_copy(iidx_hbm.at[pl.ds(base, BPW)], iidx_v, xsem).wait()
    pltpu.async_copy(utail_hbm, utail_v, xsem).wait()
    pltpu.async_copy(itail_hbm, itail_v, xsem).wait()

    lane = lax.broadcasted_iota(jnp.int32, (L,), 0)

    def stage_and_extract(tab_hbm, tail_v, idx_v, pan_v, g):
        rvec = idx_v[pl.ds(g * GB, L)]
        for l in range(GB):
            r = rvec[l]
            q = lax.shift_right_logical(r, 7)
            q = jnp.where(r >= RTAIL, 0, q)
            qoff = pl.multiple_of(q * W, W)
            for s in range(D // 8):
                pltpu.async_copy(
                    tab_hbm.at[pl.ds(s * 8, 8), pl.ds(qoff, W)],
                    buf_v.at[l, pl.ds(s * 8, 8)], dsem)

        for l in range(GB):
            pltpu.make_async_copy(tab_hbm.at[:, pl.ds(0, W)],
                                  buf_v.at[l], dsem).wait()

        wvec = jnp.bitwise_and(rvec, W - 1)
        tmask = rvec >= RTAIL
        tcol = jnp.clip(rvec - TAIL0, 0, W - 1)
        for c in range(D):
            cvec = jnp.full((L,), c, jnp.int32)
            win = plsc.load_gather(buf_v, [lane, cvec, wvec])
            tl = plsc.load_gather(tail_v, [cvec, tcol])
            pan_v[c, pl.ds(0, L)] = jnp.where(tmask, tl, win)

    def group_body(g, carry):
        stage_and_extract(ut_hbm, utail_v, uidx_v, upan_v, g)
        stage_and_extract(it_hbm, itail_v, iidx_v, ipan_v, g)
        acc = jnp.zeros((L,), jnp.float32)
        for c in range(D):
            acc = acc + upan_v[c, pl.ds(0, L)] * ipan_v[c, pl.ds(0, L)]
        out_v[pl.ds(g * GB, L)] = acc
        return carry

    lax.fori_loop(0, NG, group_body, 0)
    pltpu.sync_copy(out_v, out_hbm.at[pl.ds(base, BPW)])


def kernel(x, uid_table, iid_table):
    uidx = x[:, 0]
    iidx = x[:, 1]
    utail = lax.slice(uid_table, (TAIL0, 0), (TAIL0 + W, D)).T
    itail = lax.slice(iid_table, (TAIL0, 0), (TAIL0 + W, D)).T
    return _mf_score(uidx, iidx, uid_table.T, iid_table.T, utail, itail)


# speedup vs baseline: 1.0703x; 1.0117x over previous
import functools

import jax
import jax.numpy as jnp
from jax import lax
from jax.experimental import pallas as pl
from jax.experimental.pallas import tpu as pltpu
from jax.experimental.pallas import tpu_sc as plsc

B = 16384
D = 32
L = 16
NC, NS = 2, 16
NW = NC * NS
BPW = B // NW
GB = 16
NG = BPW // GB
W = 128
TAIL0 = 999872
RTAIL = TAIL0 + 64

_mesh = plsc.VectorSubcoreMesh(core_axis_name="c", subcore_axis_name="s")


@functools.partial(
    pl.kernel,
    mesh=_mesh,
    compiler_params=pltpu.CompilerParams(needs_layout_passes=False),
    out_type=jax.ShapeDtypeStruct((B,), jnp.float32),
    scratch_types=[
        pltpu.VMEM((BPW,), jnp.int32),
        pltpu.VMEM((BPW,), jnp.int32),
        pltpu.VMEM((GB, D, W), jnp.float32),
        pltpu.VMEM((D, W), jnp.float32),
        pltpu.VMEM((D, W), jnp.float32),
        pltpu.VMEM((D, L), jnp.float32),
        pltpu.VMEM((D, L), jnp.float32),
        pltpu.VMEM((BPW,), jnp.float32),
        pltpu.SemaphoreType.DMA,
        pltpu.SemaphoreType.DMA,
    ],
)
def _mf_score(uidx_hbm, iidx_hbm, ut_hbm, it_hbm, utail_hbm, itail_hbm,
              out_hbm, uidx_v, iidx_v, buf_v, utail_v,
              itail_v, upan_v, ipan_v, out_v, dsem, xsem):
    wid = lax.axis_index("s") * NC + lax.axis_index("c")
    base = wid * BPW

    pltpu.async_copy(uidx_hbm.at[pl.ds(base, BPW)], uidx_v, xsem).wait()
    pltpu.async_copy(iidx_hbm.at[pl.ds(base, BPW)], iidx_v, xsem).wait()
    pltpu.async_copy(utail_hbm, utail_v, xsem).wait()
    pltpu.async_copy(itail_hbm, itail_v, xsem).wait()

    lane = lax.broadcasted_iota(jnp.int32, (L,), 0)

    def stage_and_extract(tab_hbm, tail_v, idx_v, pan_v, g):
        rvec = idx_v[pl.ds(g * GB, L)]
        for l in range(GB):
            r = rvec[l]
            q = lax.shift_right_logical(r, 7)
            q = jnp.where(r >= RTAIL, 0, q)
            qoff = pl.multiple_of(q * W, W)
            pltpu.async_copy(tab_hbm.at[:, pl.ds(qoff, W)],
                             buf_v.at[l], dsem)

        for l in range(GB):
            pltpu.make_async_copy(tab_hbm.at[:, pl.ds(0, W)],
                                  buf_v.at[l], dsem).wait()

        wvec = jnp.bitwise_and(rvec, W - 1)
        tmask = rvec >= RTAIL
        tcol = jnp.clip(rvec - TAIL0, 0, W - 1)
        for c in range(D):
            cvec = jnp.full((L,), c, jnp.int32)
            win = plsc.load_gather(buf_v, [lane, cvec, wvec])
            tl = plsc.load_gather(tail_v, [cvec, tcol])
            pan_v[c, pl.ds(0, L)] = jnp.where(tmask, tl, win)

    def group_body(g, carry):
        stage_and_extract(ut_hbm, utail_v, uidx_v, upan_v, g)
        stage_and_extract(it_hbm, itail_v, iidx_v, ipan_v, g)
        acc = jnp.zeros((L,), jnp.float32)
        for c in range(D):
            acc = acc + upan_v[c, pl.ds(0, L)] * ipan_v[c, pl.ds(0, L)]
        out_v[pl.ds(g * GB, L)] = acc
        return carry

    lax.fori_loop(0, NG, group_body, 0)
    pltpu.sync_copy(out_v, out_hbm.at[pl.ds(base, BPW)])


def kernel(x, uid_table, iid_table):
    uidx = x[:, 0]
    iidx = x[:, 1]
    utail = lax.slice(uid_table, (TAIL0, 0), (TAIL0 + W, D)).T
    itail = lax.slice(iid_table, (TAIL0, 0), (TAIL0 + W, D)).T
    return _mf_score(uidx, iidx, uid_table.T, iid_table.T, utail, itail)
